# baseline (speedup 1.0000x reference)
import functools
import math

import jax
import jax.numpy as jnp
from jax import lax
from jax.experimental import pallas as pl
from jax.experimental.pallas import tpu as pltpu
from jax.experimental.pallas import tpu_sc as plsc

D_MODEL = 64
SCALE = math.sqrt(D_MODEL)

_INFO = plsc.get_sparse_core_info()
NUM_WORKERS = _INFO.num_cores * _INFO.num_subcores
LANES = _INFO.num_lanes

CHUNK = 128
NBUF = 4
PREP_COLS = 1024


def _tc_prep(table_t, vocab):
    n_blocks = (vocab + PREP_COLS - 1) // PREP_COLS

    def body(tt_ref, o_ref):
        blk = tt_ref[...] * SCALE
        a = jnp.transpose(blk[:, : PREP_COLS // 2], (1, 0))
        b = jnp.transpose(blk[:, PREP_COLS // 2 :], (1, 0))
        o_ref[...] = jnp.concatenate([a, b], axis=1)

    return pl.pallas_call(
        body,
        out_shape=jax.ShapeDtypeStruct((n_blocks * 512, 2 * D_MODEL), jnp.float32),
        grid=(n_blocks,),
        in_specs=[pl.BlockSpec((D_MODEL, PREP_COLS), lambda j: (0, j))],
        out_specs=pl.BlockSpec((512, 2 * D_MODEL), lambda j: (j, 0)),
    )(table_t)


def _make_sc_gather(n_seq, n_rows):
    mesh = plsc.VectorSubcoreMesh(core_axis_name="c", subcore_axis_name="s")

    @functools.partial(
        pl.kernel,
        out_type=jax.ShapeDtypeStruct(
            (n_seq, NUM_WORKERS, CHUNK // 2, 2 * D_MODEL), jnp.float32
        ),
        mesh=mesh,
        compiler_params=pltpu.CompilerParams(use_tc_tiling_on_sc=False),
        scratch_types=[
            pltpu.VMEM((n_seq, CHUNK), jnp.int32),
            [pltpu.VMEM((CHUNK, D_MODEL), jnp.float32) for _ in range(NBUF)],
            [pltpu.SemaphoreType.DMA for _ in range(NBUF)],
            [pltpu.SemaphoreType.DMA for _ in range(NBUF)],
        ],
    )
    def gather(xp_hbm, table_hbm, out_hbm, idx_v, rows, g_sems, o_sems):
        wid = lax.axis_index("s") * _INFO.num_cores + lax.axis_index("c")

        pltpu.sync_copy(xp_hbm.at[:, wid, :], idx_v)

        def start_gather(j, b):
            pltpu.async_copy(table_hbm.at[idx_v.at[j]], rows[b], g_sems[b])

        def out_copies(j, b):
            h = CHUNK // 2
            return [
                pltpu.make_async_copy(
                    rows[b].at[pl.ds(k * h, h), :],
                    out_hbm.at[j, wid, :, pl.ds(k * D_MODEL, D_MODEL)],
                    o_sems[b],
                )
                for k in range(2)
            ]

        for b in range(NBUF):
            start_gather(b, b)

        @pl.loop(0, n_seq, step=NBUF)
        def _chunk_loop(j):
            for b in range(NBUF):
                jj = j + b
                pltpu.make_async_copy(
                    table_hbm.at[idx_v.at[jj]], rows[b], g_sems[b]
                ).wait()

                for c in out_copies(jj, b):
                    c.start()

                bp = (b - 1) % NBUF
                jp = jj - 1

                @pl.when((jp >= 0) & (jp + NBUF < n_seq))
                def _():
                    for c in out_copies(jp, bp):
                        c.wait()
                    start_gather(jp + NBUF, bp)

        for b in range(NBUF):
            for c in out_copies(n_seq - NBUF + b, b):
                c.wait()

    return gather


def _tc_finish(lin, n_seq, n_batch):
    def body(l_ref, o_ref):
        blk = l_ref[0]
        per_s = []
        for s in range(8):
            a = jnp.transpose(blk[:, :, 8 * s : 8 * s + 8], (0, 2, 1))
            b = jnp.transpose(blk[:, :, 64 + 8 * s : 64 + 8 * s + 8], (0, 2, 1))
            per_s.append(jnp.concatenate([a, b], axis=2))
        o_ref[...] = jnp.stack(per_s, axis=0)

    return pl.pallas_call(
        body,
        out_shape=jax.ShapeDtypeStruct(
            (n_seq * 8, NUM_WORKERS, 8, CHUNK), jnp.float32
        ),
        grid=(n_seq,),
        in_specs=[
            pl.BlockSpec((1, NUM_WORKERS, D_MODEL, CHUNK), lambda j: (j, 0, 0, 0))
        ],
        out_specs=pl.BlockSpec((8, NUM_WORKERS, 8, CHUNK), lambda j: (j, 0, 0, 0)),
    )(lin)


def kernel(x, table):
    n_batch, n_seq = x.shape
    vocab = table.shape[0]

    packed = _tc_prep(table.T, vocab)
    t_rm = packed.reshape(-1, D_MODEL)

    r = x.T.astype(jnp.uint32)
    q = ((r >> 10) << 10) | ((r & 511) << 1) | ((r >> 9) & 1)
    xp = q.astype(jnp.int32).reshape(n_seq, NUM_WORKERS, CHUNK)

    lin = _make_sc_gather(n_seq, t_rm.shape[0])(xp, t_rm)

    out4 = _tc_finish(lin, n_seq, n_batch)
    o5 = out4.reshape(n_seq, 8, NUM_WORKERS, 8, CHUNK)
    return o5.transpose(2, 4, 0, 1, 3).reshape(n_batch, n_seq, D_MODEL)


# speedup vs baseline: 3.8402x; 3.8402x over previous
import functools
import math

import jax
import jax.numpy as jnp
from jax import lax
from jax.experimental import pallas as pl
from jax.experimental.pallas import tpu as pltpu
from jax.experimental.pallas import tpu_sc as plsc

D_MODEL = 64
SCALE = math.sqrt(D_MODEL)

_INFO = plsc.get_sparse_core_info()
NUM_WORKERS = _INFO.num_cores * _INFO.num_subcores
LANES = _INFO.num_lanes

CHUNK = 128
NBUF = 4
PREP_COLS = 4096
FIN_J = 4


def _tc_prep(table_t, vocab):
    n_blocks = (vocab + PREP_COLS - 1) // PREP_COLS

    def body(tt_ref, o_ref):
        blk = tt_ref[...] * SCALE
        a = jnp.transpose(blk[:, : PREP_COLS // 2], (1, 0))
        b = jnp.transpose(blk[:, PREP_COLS // 2 :], (1, 0))
        o_ref[...] = jnp.concatenate([a, b], axis=1)

    return pl.pallas_call(
        body,
        out_shape=jax.ShapeDtypeStruct(
            (n_blocks * PREP_COLS // 2, 2 * D_MODEL), jnp.float32
        ),
        grid=(n_blocks,),
        in_specs=[pl.BlockSpec((D_MODEL, PREP_COLS), lambda j: (0, j))],
        out_specs=pl.BlockSpec((PREP_COLS // 2, 2 * D_MODEL), lambda j: (j, 0)),
    )(table_t)


def _make_sc_gather(n_seq, n_rows):
    mesh = plsc.VectorSubcoreMesh(core_axis_name="c", subcore_axis_name="s")

    @functools.partial(
        pl.kernel,
        out_type=jax.ShapeDtypeStruct(
            (n_seq, NUM_WORKERS, CHUNK // 2, 2 * D_MODEL), jnp.float32
        ),
        mesh=mesh,
        compiler_params=pltpu.CompilerParams(use_tc_tiling_on_sc=False),
        scratch_types=[
            pltpu.VMEM((n_seq, CHUNK), jnp.int32),
            [pltpu.VMEM((CHUNK, D_MODEL), jnp.float32) for _ in range(NBUF)],
            [pltpu.SemaphoreType.DMA for _ in range(NBUF)],
            [pltpu.SemaphoreType.DMA for _ in range(NBUF)],
        ],
    )
    def gather(xp_hbm, table_hbm, out_hbm, idx_v, rows, g_sems, o_sems):
        wid = lax.axis_index("s") * _INFO.num_cores + lax.axis_index("c")

        pltpu.sync_copy(xp_hbm.at[:, wid, :], idx_v)

        def start_gather(j, b):
            pltpu.async_copy(table_hbm.at[idx_v.at[j]], rows[b], g_sems[b])

        def out_copies(j, b):
            h = CHUNK // 2
            return [
                pltpu.make_async_copy(
                    rows[b].at[pl.ds(k * h, h), :],
                    out_hbm.at[j, wid, :, pl.ds(k * D_MODEL, D_MODEL)],
                    o_sems[b],
                )
                for k in range(2)
            ]

        for b in range(NBUF):
            start_gather(b, b)

        @pl.loop(0, n_seq, step=NBUF)
        def _chunk_loop(j):
            for b in range(NBUF):
                jj = j + b
                pltpu.make_async_copy(
                    table_hbm.at[idx_v.at[jj]], rows[b], g_sems[b]
                ).wait()

                for c in out_copies(jj, b):
                    c.start()

                bp = (b - 1) % NBUF
                jp = jj - 1

                @pl.when((jp >= 0) & (jp + NBUF < n_seq))
                def _():
                    for c in out_copies(jp, bp):
                        c.wait()
                    start_gather(jp + NBUF, bp)

        for b in range(NBUF):
            for c in out_copies(n_seq - NBUF + b, b):
                c.wait()

    return gather


def _tc_finish(lin, n_seq, n_batch):
    def body(l_ref, o_ref):
        blk = l_ref[...]
        t1 = jnp.transpose(blk[:, :, :, :64], (0, 1, 3, 2))
        t2 = jnp.transpose(blk[:, :, :, 64:], (0, 1, 3, 2))
        cc = jnp.concatenate([t1, t2], axis=3)
        o_ref[...] = jnp.transpose(
            cc.reshape(FIN_J, 32, 8, 8, 128), (0, 2, 1, 3, 4)
        ).reshape(FIN_J * 8, NUM_WORKERS, 8, CHUNK)

    return pl.pallas_call(
        body,
        out_shape=jax.ShapeDtypeStruct(
            (n_seq * 8, NUM_WORKERS, 8, CHUNK), jnp.float32
        ),
        grid=(n_seq // FIN_J,),
        in_specs=[
            pl.BlockSpec(
                (FIN_J, NUM_WORKERS, D_MODEL, CHUNK), lambda j: (j, 0, 0, 0)
            )
        ],
        out_specs=pl.BlockSpec(
            (FIN_J * 8, NUM_WORKERS, 8, CHUNK), lambda j: (j, 0, 0, 0)
        ),
    )(lin)


def kernel(x, table):
    n_batch, n_seq = x.shape
    vocab = table.shape[0]

    packed = _tc_prep(table.T, vocab)
    t_rm = packed.reshape(-1, D_MODEL)

    sh = PREP_COLS.bit_length() - 1
    half_mask = PREP_COLS // 2 - 1
    r = x.T.astype(jnp.uint32)
    q = ((r >> sh) << sh) | ((r & half_mask) << 1) | ((r >> (sh - 1)) & 1)
    xp = q.astype(jnp.int32).reshape(n_seq, NUM_WORKERS, CHUNK)

    lin = _make_sc_gather(n_seq, t_rm.shape[0])(xp, t_rm)

    out4 = _tc_finish(lin, n_seq, n_batch)
    o5 = out4.reshape(n_seq, 8, NUM_WORKERS, 8, CHUNK)
    return o5.transpose(2, 4, 0, 1, 3).reshape(n_batch, n_seq, D_MODEL)


# speedup vs baseline: 4.2576x; 1.1087x over previous
import functools
import math

import jax
import jax.numpy as jnp
from jax import lax
from jax.experimental import pallas as pl
from jax.experimental.pallas import tpu as pltpu
from jax.experimental.pallas import tpu_sc as plsc

D_MODEL = 64
SCALE = math.sqrt(D_MODEL)

_INFO = plsc.get_sparse_core_info()
NUM_WORKERS = _INFO.num_cores * _INFO.num_subcores
LANES = _INFO.num_lanes

CHUNK = 128
NBUF = 4
PREP_COLS = 8192
FIN_J = 4


def _tc_prep(table_t, vocab):
    n_blocks = (vocab + PREP_COLS - 1) // PREP_COLS

    def body(tt_ref, o_ref):
        blk = tt_ref[...] * SCALE
        a = jnp.transpose(blk[:, : PREP_COLS // 2], (1, 0))
        b = jnp.transpose(blk[:, PREP_COLS // 2 :], (1, 0))
        o_ref[...] = jnp.concatenate([a, b], axis=1)

    return pl.pallas_call(
        body,
        out_shape=jax.ShapeDtypeStruct(
            (n_blocks * PREP_COLS // 2, 2 * D_MODEL), jnp.float32
        ),
        grid=(n_blocks,),
        in_specs=[pl.BlockSpec((D_MODEL, PREP_COLS), lambda j: (0, j))],
        out_specs=pl.BlockSpec((PREP_COLS // 2, 2 * D_MODEL), lambda j: (j, 0)),
    )(table_t)


def _make_sc_gather(n_seq, n_rows):
    mesh = plsc.VectorSubcoreMesh(core_axis_name="c", subcore_axis_name="s")

    @functools.partial(
        pl.kernel,
        out_type=jax.ShapeDtypeStruct(
            (n_seq, NUM_WORKERS, CHUNK // 2, 2 * D_MODEL), jnp.float32
        ),
        mesh=mesh,
        compiler_params=pltpu.CompilerParams(use_tc_tiling_on_sc=False),
        scratch_types=[
            pltpu.VMEM((n_seq, CHUNK), jnp.int32),
            [pltpu.VMEM((CHUNK, D_MODEL), jnp.float32) for _ in range(NBUF)],
            [pltpu.SemaphoreType.DMA for _ in range(NBUF)],
            [pltpu.SemaphoreType.DMA for _ in range(NBUF)],
        ],
    )
    def gather(xp_hbm, table_hbm, out_hbm, idx_v, rows, g_sems, o_sems):
        wid = lax.axis_index("s") * _INFO.num_cores + lax.axis_index("c")

        pltpu.sync_copy(xp_hbm.at[:, wid, :], idx_v)

        def start_gather(j, b):
            pltpu.async_copy(table_hbm.at[idx_v.at[j]], rows[b], g_sems[b])

        def out_copies(j, b):
            h = CHUNK // 2
            return [
                pltpu.make_async_copy(
                    rows[b].at[pl.ds(k * h, h), :],
                    out_hbm.at[j, wid, :, pl.ds(k * D_MODEL, D_MODEL)],
                    o_sems[b],
                )
                for k in range(2)
            ]

        for b in range(NBUF):
            start_gather(b, b)

        @pl.loop(0, n_seq, step=NBUF)
        def _chunk_loop(j):
            for b in range(NBUF):
                jj = j + b
                pltpu.make_async_copy(
                    table_hbm.at[idx_v.at[jj]], rows[b], g_sems[b]
                ).wait()

                for c in out_copies(jj, b):
                    c.start()

                bp = (b - 1) % NBUF
                jp = jj - 1

                @pl.when((jp >= 0) & (jp + NBUF < n_seq))
                def _():
                    for c in out_copies(jp, bp):
                        c.wait()
                    start_gather(jp + NBUF, bp)

        for b in range(NBUF):
            for c in out_copies(n_seq - NBUF + b, b):
                c.wait()

    return gather


def _tc_finish(lin, n_seq, n_batch):
    def body(l_ref, o_ref):
        blk = l_ref[...]
        t1 = jnp.transpose(blk[:, :, :, :64], (0, 1, 3, 2))
        t2 = jnp.transpose(blk[:, :, :, 64:], (0, 1, 3, 2))
        cc = jnp.concatenate([t1, t2], axis=3)
        o_ref[...] = jnp.transpose(
            cc.reshape(FIN_J, 32, 8, 8, 128), (0, 2, 1, 3, 4)
        ).reshape(FIN_J * 8, NUM_WORKERS, 8, CHUNK)

    return pl.pallas_call(
        body,
        out_shape=jax.ShapeDtypeStruct(
            (n_seq * 8, NUM_WORKERS, 8, CHUNK), jnp.float32
        ),
        grid=(n_seq // FIN_J,),
        in_specs=[
            pl.BlockSpec(
                (FIN_J, NUM_WORKERS, D_MODEL, CHUNK), lambda j: (j, 0, 0, 0)
            )
        ],
        out_specs=pl.BlockSpec(
            (FIN_J * 8, NUM_WORKERS, 8, CHUNK), lambda j: (j, 0, 0, 0)
        ),
    )(lin)


def kernel(x, table):
    n_batch, n_seq = x.shape
    vocab = table.shape[0]

    packed = _tc_prep(table.T, vocab)
    t_rm = packed.reshape(-1, D_MODEL)

    sh = PREP_COLS.bit_length() - 1
    half_mask = PREP_COLS // 2 - 1
    r = x.T.astype(jnp.uint32)
    q = ((r >> sh) << sh) | ((r & half_mask) << 1) | ((r >> (sh - 1)) & 1)
    xp = q.astype(jnp.int32).reshape(n_seq, NUM_WORKERS, CHUNK)

    lin = _make_sc_gather(n_seq, t_rm.shape[0])(xp, t_rm)

    out4 = _tc_finish(lin, n_seq, n_batch)
    o5 = out4.reshape(n_seq, 8, NUM_WORKERS, 8, CHUNK)
    return o5.transpose(2, 4, 0, 1, 3).reshape(n_batch, n_seq, D_MODEL)


# speedup vs baseline: 5.1492x; 1.2094x over previous
import functools
import math

import jax
import jax.numpy as jnp
from jax import lax
from jax.experimental import pallas as pl
from jax.experimental.pallas import tpu as pltpu
from jax.experimental.pallas import tpu_sc as plsc

D_MODEL = 64
SCALE = math.sqrt(D_MODEL)

_INFO = plsc.get_sparse_core_info()
NUM_WORKERS = _INFO.num_cores * _INFO.num_subcores
LANES = _INFO.num_lanes

CHUNK = 128
NBUF = 4
PREP_COLS = 16384
FIN_J = 8


def _tc_prep(table_t, vocab):
    n_blocks = (vocab + PREP_COLS - 1) // PREP_COLS

    def body(tt_ref, o_ref):
        blk = tt_ref[...] * SCALE
        a = jnp.transpose(blk[:, : PREP_COLS // 2], (1, 0))
        b = jnp.transpose(blk[:, PREP_COLS // 2 :], (1, 0))
        o_ref[...] = jnp.concatenate([a, b], axis=1)

    return pl.pallas_call(
        body,
        out_shape=jax.ShapeDtypeStruct(
            (n_blocks * PREP_COLS // 2, 2 * D_MODEL), jnp.float32
        ),
        grid=(n_blocks,),
        in_specs=[pl.BlockSpec((D_MODEL, PREP_COLS), lambda j: (0, j))],
        out_specs=pl.BlockSpec((PREP_COLS // 2, 2 * D_MODEL), lambda j: (j, 0)),
    )(table_t)


def _make_sc_gather(n_seq, n_rows):
    mesh = plsc.VectorSubcoreMesh(core_axis_name="c", subcore_axis_name="s")

    @functools.partial(
        pl.kernel,
        out_type=jax.ShapeDtypeStruct(
            (n_seq, NUM_WORKERS, CHUNK // 2, 2 * D_MODEL), jnp.float32
        ),
        mesh=mesh,
        compiler_params=pltpu.CompilerParams(use_tc_tiling_on_sc=False),
        scratch_types=[
            pltpu.VMEM((n_seq, CHUNK), jnp.int32),
            [pltpu.VMEM((CHUNK, D_MODEL), jnp.float32) for _ in range(NBUF)],
            [pltpu.SemaphoreType.DMA for _ in range(NBUF)],
            [pltpu.SemaphoreType.DMA for _ in range(NBUF)],
        ],
    )
    def gather(xp_hbm, table_hbm, out_hbm, idx_v, rows, g_sems, o_sems):
        wid = lax.axis_index("s") * _INFO.num_cores + lax.axis_index("c")

        pltpu.sync_copy(xp_hbm.at[:, wid, :], idx_v)

        def start_gather(j, b):
            pltpu.async_copy(table_hbm.at[idx_v.at[j]], rows[b], g_sems[b])

        def out_copies(j, b):
            h = CHUNK // 2
            return [
                pltpu.make_async_copy(
                    rows[b].at[pl.ds(k * h, h), :],
                    out_hbm.at[j, wid, :, pl.ds(k * D_MODEL, D_MODEL)],
                    o_sems[b],
                )
                for k in range(2)
            ]

        for b in range(NBUF):
            start_gather(b, b)

        @pl.loop(0, n_seq, step=NBUF)
        def _chunk_loop(j):
            for b in range(NBUF):
                jj = j + b
                pltpu.make_async_copy(
                    table_hbm.at[idx_v.at[jj]], rows[b], g_sems[b]
                ).wait()

                for c in out_copies(jj, b):
                    c.start()

                bp = (b - 1) % NBUF
                jp = jj - 1

                @pl.when((jp >= 0) & (jp + NBUF < n_seq))
                def _():
                    for c in out_copies(jp, bp):
                        c.wait()
                    start_gather(jp + NBUF, bp)

        for b in range(NBUF):
            for c in out_copies(n_seq - NBUF + b, b):
                c.wait()

    return gather


def _tc_finish(lin, n_seq, n_batch):
    def body(l_ref, o_ref):
        blk = l_ref[...]
        t1 = jnp.transpose(blk[:, :, :, :64], (0, 1, 3, 2))
        t2 = jnp.transpose(blk[:, :, :, 64:], (0, 1, 3, 2))
        cc = jnp.concatenate([t1, t2], axis=3)
        o_ref[...] = jnp.transpose(
            cc.reshape(FIN_J, 32, 8, 8, 128), (0, 2, 1, 3, 4)
        ).reshape(FIN_J * 8, NUM_WORKERS, 8, CHUNK)

    return pl.pallas_call(
        body,
        out_shape=jax.ShapeDtypeStruct(
            (n_seq * 8, NUM_WORKERS, 8, CHUNK), jnp.float32
        ),
        grid=(n_seq // FIN_J,),
        in_specs=[
            pl.BlockSpec(
                (FIN_J, NUM_WORKERS, D_MODEL, CHUNK), lambda j: (j, 0, 0, 0)
            )
        ],
        out_specs=pl.BlockSpec(
            (FIN_J * 8, NUM_WORKERS, 8, CHUNK), lambda j: (j, 0, 0, 0)
        ),
    )(lin)


def kernel(x, table):
    n_batch, n_seq = x.shape
    vocab = table.shape[0]

    packed = _tc_prep(table.T, vocab)
    t_rm = packed.reshape(-1, D_MODEL)

    sh = PREP_COLS.bit_length() - 1
    half_mask = PREP_COLS // 2 - 1
    r = x.T.astype(jnp.uint32)
    q = ((r >> sh) << sh) | ((r & half_mask) << 1) | ((r >> (sh - 1)) & 1)
    xp = q.astype(jnp.int32).reshape(n_seq, NUM_WORKERS, CHUNK)

    lin = _make_sc_gather(n_seq, t_rm.shape[0])(xp, t_rm)

    out4 = _tc_finish(lin, n_seq, n_batch)
    o5 = out4.reshape(n_seq, 8, NUM_WORKERS, 8, CHUNK)
    return o5.transpose(2, 4, 0, 1, 3).reshape(n_batch, n_seq, D_MODEL)
